# Initial kernel scaffold; baseline (speedup 1.0000x reference)
import jax
import jax.numpy as jnp
from jax.experimental import pallas as pl


def kernel(x, W1, b1, W2, b2):
    raise NotImplementedError("write your pallas kernel here")


# speedup vs baseline: 1.0187x; 1.0187x over previous
import functools

import jax
import jax.numpy as jnp
import numpy as np
from jax import lax
from jax.experimental import pallas as pl
from jax.experimental.pallas import tpu as pltpu
from jax.experimental.pallas import tpu_sc as plsc

B, S, D, H = 4, 4096, 2048, 512
K = S // 2

BSZ = 512


def _score_body(x_ref, w1t_ref, b1_ref, w2_ref, b2_ref, o_ref):
    xb = x_ref[...].astype(jnp.bfloat16)
    h = lax.dot_general(
        xb, w1t_ref[...], (((1,), (0,)), ((), ())),
        preferred_element_type=jnp.float32)
    h = jnp.maximum(h + b1_ref[...], 0.0)
    hb = h.astype(jnp.bfloat16)
    s = lax.dot_general(hb, w2_ref[...], (((1,), (1,)), ((), ())),
                        preferred_element_type=jnp.float32)
    o_ref[...] = (s[:, 0:1] + b2_ref[0, 0])[None]


def _scores_tc(x2d, W1t, b1, W2, b2):
    nblk = (B * S) // BSZ
    out = pl.pallas_call(
        _score_body,
        grid=(nblk,),
        in_specs=[
            pl.BlockSpec((BSZ, D), lambda i: (i, 0)),
            pl.BlockSpec((D, H), lambda i: (0, 0)),
            pl.BlockSpec((1, H), lambda i: (0, 0)),
            pl.BlockSpec((128, H), lambda i: (0, 0)),
            pl.BlockSpec((1, 1), lambda i: (0, 0)),
        ],
        out_specs=pl.BlockSpec((1, BSZ, 1), lambda i: (i, 0, 0)),
        out_shape=jax.ShapeDtypeStruct((nblk, BSZ, 1), jnp.float32),
    )(x2d, W1t, b1, W2, b2)
    return out.reshape(B, S)


NC, NS, L = 2, 16, 16
NW = NC * NS
SLOTS = NW // B
RPT = K // SLOTS
CH = 16
NCHUNK = RPT // CH
_MIN32 = np.int32(-(2 ** 31))


def _sc_body(scores_hbm, x_hbm, outx_hbm, outs_hbm,
             scores_v, keys_v, idx_v, outs_v, gbuf0, gbuf1,
             gsem0, gsem1, osem0, osem1):
    c = lax.axis_index("c")
    s = lax.axis_index("s")
    wid = s * NC + c
    b = wid // SLOTS
    slot = wid % SLOTS

    pltpu.sync_copy(scores_hbm.at[b], scores_v)

    def kbody(i, carry):
        sv = scores_v[pl.ds(i * L, L)]
        bits = lax.bitcast_convert_type(sv, jnp.int32)
        key = bits ^ (lax.shift_right_arithmetic(bits, 31)
                      & jnp.int32(0x7FFFFFFF))
        key = jnp.where(key == jnp.int32(-1), jnp.int32(0), key)
        keys_v[pl.ds(i * L, L)] = key
        return carry
    lax.fori_loop(0, S // L, kbody, jnp.int32(0))

    def count_ge(scand):
        def cb(i, cv):
            kv = keys_v[pl.ds(i * L, L)]
            return cv + jnp.where(kv >= scand, 1, 0).astype(jnp.int32)
        return jnp.sum(lax.fori_loop(0, S // L, cb,
                                     jnp.zeros((L,), jnp.int32)))

    def sbody(j, t):
        cand = t | lax.shift_left(jnp.int32(1), jnp.int32(31) - j)
        cnt = count_ge(cand ^ _MIN32)
        return jnp.where(cnt >= K, cand, t)
    tbits = lax.fori_loop(0, 32, sbody, jnp.int32(0))
    thr = tbits ^ _MIN32

    def gtb(i, cv):
        kv = keys_v[pl.ds(i * L, L)]
        return cv + jnp.where(kv > thr, 1, 0).astype(jnp.int32)
    cgt = jnp.sum(lax.fori_loop(0, S // L, gtb, jnp.zeros((L,), jnp.int32)))
    need = K - cgt

    lanes = lax.iota(jnp.int32, L)

    def cbody(i, carry):
        off, eqs = carry
        kv = keys_v[pl.ds(i * L, L)]
        gt = kv > thr
        eq = kv == thr
        eqc = plsc.cumsum(jnp.where(eq, 1, 0).astype(jnp.int32))
        keep = gt | (eq & ((eqs + eqc - 1) < need))
        kc = plsc.cumsum(jnp.where(keep, 1, 0).astype(jnp.int32))
        pos = off + kc - 1
        plsc.store_scatter(idx_v, [pos], lanes + i * L, mask=keep)
        return (off + jnp.sum(jnp.where(keep, 1, 0).astype(jnp.int32)),
                eqs + jnp.sum(jnp.where(eq, 1, 0).astype(jnp.int32)))
    lax.fori_loop(0, S // L, cbody, (jnp.int32(0), jnp.int32(0)))

    @pl.when(slot == 0)
    def _():
        def sgb(i, carry):
            iv = idx_v[pl.ds(i * L, L)]
            outs_v[pl.ds(i * L, L)] = plsc.load_gather(scores_v, [iv])
            return carry
        lax.fori_loop(0, K // L, sgb, jnp.int32(0))
        pltpu.sync_copy(outs_v, outs_hbm.at[b])

    base = slot * RPT
    out0 = b * K + base
    row_off = b * S
    gbufs = (gbuf0, gbuf1)
    gsems = (gsem0, gsem1)
    osems = (osem0, osem1)
    pend_g = [None, None]
    pend_o = [None, None]
    for t in range(NCHUNK):
        bi = t % 2
        if pend_o[bi] is not None:
            pend_o[bi].wait()
        giv = idx_v[pl.ds(base + t * CH, CH)] + row_off
        pend_g[bi] = pltpu.async_copy(x_hbm.at[giv], gbufs[bi], gsems[bi])
        if t >= 1:
            pj = (t - 1) % 2
            pend_g[pj].wait()
            pend_o[pj] = pltpu.async_copy(
                gbufs[pj], outx_hbm.at[pl.ds(out0 + (t - 1) * CH, CH)],
                osems[pj])
    lb = (NCHUNK - 1) % 2
    pend_g[lb].wait()
    pend_o[lb] = pltpu.async_copy(
        gbufs[lb], outx_hbm.at[pl.ds(out0 + (NCHUNK - 1) * CH, CH)], osems[lb])
    pend_o[0].wait()
    pend_o[1].wait()


@functools.cache
def _sc_topk_gather():
    return pl.kernel(
        _sc_body,
        out_type=(jax.ShapeDtypeStruct((B * K, D), jnp.float32),
                  jax.ShapeDtypeStruct((B, K), jnp.float32)),
        mesh=plsc.VectorSubcoreMesh(core_axis_name="c", subcore_axis_name="s"),
        compiler_params=pltpu.CompilerParams(needs_layout_passes=False),
        scratch_types=[
            pltpu.VMEM((S,), jnp.float32),
            pltpu.VMEM((S,), jnp.int32),
            pltpu.VMEM((K,), jnp.int32),
            pltpu.VMEM((K,), jnp.float32),
            pltpu.VMEM((CH, D), jnp.float32),
            pltpu.VMEM((CH, D), jnp.float32),
            pltpu.SemaphoreType.DMA,
            pltpu.SemaphoreType.DMA,
            pltpu.SemaphoreType.DMA,
            pltpu.SemaphoreType.DMA,
        ],
    )


def kernel(x, W1, b1, W2, b2):
    x2d = x.reshape(B * S, D)
    W1tb = W1.T.astype(jnp.bfloat16)
    W2pb = jnp.pad(W2, ((0, 127), (0, 0))).astype(jnp.bfloat16)
    scores = _scores_tc(x2d, W1tb, b1.reshape(1, H), W2pb, b2.reshape(1, 1))
    sliced_x, sliced_scores = _sc_topk_gather()(scores, x2d)
    return sliced_x.reshape(B, K, D), sliced_scores


# speedup vs baseline: 1.1745x; 1.1530x over previous
import functools

import jax
import jax.numpy as jnp
import numpy as np
from jax import lax
from jax.experimental import pallas as pl
from jax.experimental.pallas import tpu as pltpu
from jax.experimental.pallas import tpu_sc as plsc

B, S, D, H = 4, 4096, 2048, 512
K = S // 2

BSZ = 512


def _score_body(x_ref, w1t_ref, b1_ref, w2_ref, b2_ref, o_ref):
    xb = x_ref[...].astype(jnp.bfloat16)
    h = lax.dot_general(
        xb, w1t_ref[...], (((1,), (0,)), ((), ())),
        preferred_element_type=jnp.float32)
    h = jnp.maximum(h + b1_ref[...], 0.0)
    hb = h.astype(jnp.bfloat16)
    s = lax.dot_general(hb, w2_ref[...], (((1,), (1,)), ((), ())),
                        preferred_element_type=jnp.float32)
    o_ref[...] = (s[:, 0:1] + b2_ref[0, 0])[None]


def _scores_tc(x2d, W1t, b1, W2, b2):
    nblk = (B * S) // BSZ
    out = pl.pallas_call(
        _score_body,
        grid=(nblk,),
        in_specs=[
            pl.BlockSpec((BSZ, D), lambda i: (i, 0)),
            pl.BlockSpec((D, H), lambda i: (0, 0)),
            pl.BlockSpec((1, H), lambda i: (0, 0)),
            pl.BlockSpec((128, H), lambda i: (0, 0)),
            pl.BlockSpec((1, 1), lambda i: (0, 0)),
        ],
        out_specs=pl.BlockSpec((1, BSZ, 1), lambda i: (i, 0, 0)),
        out_shape=jax.ShapeDtypeStruct((nblk, BSZ, 1), jnp.float32),
    )(x2d, W1t, b1, W2, b2)
    return out.reshape(B, S)


NC, NS, L = 2, 16, 16
NW = NC * NS
SLOTS = NW // B
RPT = K // SLOTS
CH = 16
NCHUNK = RPT // CH
_MIN32 = np.int32(-(2 ** 31))


_UNROLL = 16


def _sc_body(scores_hbm, x_hbm, outx_hbm, outs_hbm,
             scores_v, keys_v, idx_v, outs_v, gbuf0, gbuf1, gbuf2,
             gsem0, gsem1, gsem2, osem0, osem1, osem2):
    c = lax.axis_index("c")
    s = lax.axis_index("s")
    wid = s * NC + c
    b = wid // SLOTS
    slot = wid % SLOTS

    pltpu.sync_copy(scores_hbm.at[b], scores_v)

    def kbody(i, carry):
        base = i * (L * _UNROLL)
        for u in range(_UNROLL):
            sv = scores_v[pl.ds(base + u * L, L)]
            bits = lax.bitcast_convert_type(sv, jnp.int32)
            key = bits ^ (lax.shift_right_arithmetic(bits, 31)
                          & jnp.int32(0x7FFFFFFF))
            key = jnp.where(key == jnp.int32(-1), jnp.int32(0), key)
            keys_v[pl.ds(base + u * L, L)] = key
        return carry
    lax.fori_loop(0, S // (L * _UNROLL), kbody, jnp.int32(0))

    def count_ge(scand):
        def cb(i, cv):
            base = i * (L * _UNROLL)
            for u in range(_UNROLL):
                kv = keys_v[pl.ds(base + u * L, L)]
                cv = cv + jnp.where(kv >= scand, 1, 0).astype(jnp.int32)
            return cv
        return jnp.sum(lax.fori_loop(0, S // (L * _UNROLL), cb,
                                     jnp.zeros((L,), jnp.int32)))

    def sbody(j, t):
        cand = t | lax.shift_left(jnp.int32(1), jnp.int32(31) - j)
        cnt = count_ge(cand ^ _MIN32)
        return jnp.where(cnt >= K, cand, t)
    tbits = lax.fori_loop(0, 32, sbody, jnp.int32(0))
    thr = tbits ^ _MIN32

    def gtcb(i, cv):
        base = i * (L * _UNROLL)
        for u in range(_UNROLL):
            kv = keys_v[pl.ds(base + u * L, L)]
            cv = cv + jnp.where(kv > thr, 1, 0).astype(jnp.int32)
        return cv
    cgt = jnp.sum(lax.fori_loop(0, S // (L * _UNROLL), gtcb,
                                jnp.zeros((L,), jnp.int32)))
    need = K - cgt

    lanes = lax.iota(jnp.int32, L)

    def cbody(i, carry):
        off, eqs = carry
        base = i * (L * 4)
        for u in range(4):
            kv = keys_v[pl.ds(base + u * L, L)]
            gt = kv > thr
            eq = kv == thr
            eqc = plsc.cumsum(jnp.where(eq, 1, 0).astype(jnp.int32))
            keep = gt | (eq & ((eqs + eqc - 1) < need))
            kc = plsc.cumsum(jnp.where(keep, 1, 0).astype(jnp.int32))
            pos = off + kc - 1
            plsc.store_scatter(idx_v, [pos], lanes + (base + u * L), mask=keep)
            off = off + jnp.sum(jnp.where(keep, 1, 0).astype(jnp.int32))
            eqs = eqs + jnp.sum(jnp.where(eq, 1, 0).astype(jnp.int32))
        return (off, eqs)
    lax.fori_loop(0, S // (L * 4), cbody, (jnp.int32(0), jnp.int32(0)))

    @pl.when(slot == 0)
    def _():
        def sgb(i, carry):
            base = i * (L * 8)
            for u in range(8):
                iv = idx_v[pl.ds(base + u * L, L)]
                outs_v[pl.ds(base + u * L, L)] = plsc.load_gather(
                    scores_v, [iv])
            return carry
        lax.fori_loop(0, K // (L * 8), sgb, jnp.int32(0))
        pltpu.sync_copy(outs_v, outs_hbm.at[b])

    base = slot * RPT
    out0 = b * K + base
    row_off = b * S
    gbufs = (gbuf0, gbuf1, gbuf2)
    gsems = (gsem0, gsem1, gsem2)
    osems = (osem0, osem1, osem2)
    nb = 3
    pend_g = [None] * nb
    pend_o = [None] * nb
    for t in range(NCHUNK):
        bi = t % nb
        if pend_o[bi] is not None:
            pend_o[bi].wait()
        giv = idx_v[pl.ds(base + t * CH, CH)] + row_off
        pend_g[bi] = pltpu.async_copy(x_hbm.at[giv], gbufs[bi], gsems[bi])
        if t >= 1:
            pj = (t - 1) % nb
            pend_g[pj].wait()
            pend_o[pj] = pltpu.async_copy(
                gbufs[pj], outx_hbm.at[pl.ds(out0 + (t - 1) * CH, CH)],
                osems[pj])
    lb = (NCHUNK - 1) % nb
    pend_g[lb].wait()
    pend_o[lb] = pltpu.async_copy(
        gbufs[lb], outx_hbm.at[pl.ds(out0 + (NCHUNK - 1) * CH, CH)], osems[lb])
    for o in pend_o:
        if o is not None:
            o.wait()


@functools.cache
def _sc_topk_gather():
    return pl.kernel(
        _sc_body,
        out_type=(jax.ShapeDtypeStruct((B * K, D), jnp.float32),
                  jax.ShapeDtypeStruct((B, K), jnp.float32)),
        mesh=plsc.VectorSubcoreMesh(core_axis_name="c", subcore_axis_name="s"),
        compiler_params=pltpu.CompilerParams(needs_layout_passes=False),
        scratch_types=[
            pltpu.VMEM((S,), jnp.float32),
            pltpu.VMEM((S,), jnp.int32),
            pltpu.VMEM((K,), jnp.int32),
            pltpu.VMEM((K,), jnp.float32),
            pltpu.VMEM((CH, D), jnp.float32),
            pltpu.VMEM((CH, D), jnp.float32),
            pltpu.VMEM((CH, D), jnp.float32),
            pltpu.SemaphoreType.DMA,
            pltpu.SemaphoreType.DMA,
            pltpu.SemaphoreType.DMA,
            pltpu.SemaphoreType.DMA,
            pltpu.SemaphoreType.DMA,
            pltpu.SemaphoreType.DMA,
        ],
    )


def kernel(x, W1, b1, W2, b2):
    x2d = x.reshape(B * S, D)
    W1tb = W1.T.astype(jnp.bfloat16)
    W2pb = jnp.pad(W2, ((0, 127), (0, 0))).astype(jnp.bfloat16)
    scores = _scores_tc(x2d, W1tb, b1.reshape(1, H), W2pb, b2.reshape(1, 1))
    sliced_x, sliced_scores = _sc_topk_gather()(scores, x2d)
    return sliced_x.reshape(B, K, D), sliced_scores


# speedup vs baseline: 1.2423x; 1.0577x over previous
import functools

import jax
import jax.numpy as jnp
import numpy as np
from jax import lax
from jax.experimental import pallas as pl
from jax.experimental.pallas import tpu as pltpu
from jax.experimental.pallas import tpu_sc as plsc

B, S, D, H = 4, 4096, 2048, 512
K = S // 2

BSZ = 1024


def _score_body(x_ref, w1t_ref, b1_ref, w2_ref, b2_ref, o_ref):
    xb = x_ref[...].astype(jnp.bfloat16)
    h = lax.dot_general(
        xb, w1t_ref[...], (((1,), (1,)), ((), ())),
        preferred_element_type=jnp.float32)
    h = jnp.maximum(h + b1_ref[...], 0.0)
    hb = h.astype(jnp.bfloat16)
    s = lax.dot_general(hb, w2_ref[...], (((1,), (1,)), ((), ())),
                        preferred_element_type=jnp.float32)
    o_ref[...] = (s[:, 0:1] + b2_ref[0, 0])[None]


def _scores_tc(x2d, W1t, b1, W2, b2):
    nblk = (B * S) // BSZ
    out = pl.pallas_call(
        _score_body,
        grid=(nblk,),
        in_specs=[
            pl.BlockSpec((BSZ, D), lambda i: (i, 0)),
            pl.BlockSpec((H, D), lambda i: (0, 0)),
            pl.BlockSpec((1, H), lambda i: (0, 0)),
            pl.BlockSpec((128, H), lambda i: (0, 0)),
            pl.BlockSpec((1, 1), lambda i: (0, 0)),
        ],
        out_specs=pl.BlockSpec((1, BSZ, 1), lambda i: (i, 0, 0)),
        out_shape=jax.ShapeDtypeStruct((nblk, BSZ, 1), jnp.float32),
    )(x2d, W1t, b1, W2, b2)
    return out.reshape(B, S)


NC, NS, L = 2, 16, 16
NW = NC * NS
SLOTS = NW // B
RPT = K // SLOTS
CH = 16
NCHUNK = RPT // CH
_MIN32 = np.int32(-(2 ** 31))


_UNROLL = 16


def _sc_body(scores_hbm, x_hbm, outx_hbm, outs_hbm,
             scores_v, keys_v, idx_v, outs_v, gbuf0, gbuf1, gbuf2,
             gsem0, gsem1, gsem2, osem0, osem1, osem2):
    c = lax.axis_index("c")
    s = lax.axis_index("s")
    wid = s * NC + c
    b = wid // SLOTS
    slot = wid % SLOTS

    pltpu.sync_copy(scores_hbm.at[b], scores_v)

    def kbody(i, carry):
        base = i * (L * _UNROLL)
        for u in range(_UNROLL):
            sv = scores_v[pl.ds(base + u * L, L)]
            bits = lax.bitcast_convert_type(sv, jnp.int32)
            key = bits ^ (lax.shift_right_arithmetic(bits, 31)
                          & jnp.int32(0x7FFFFFFF))
            key = jnp.where(key == jnp.int32(-1), jnp.int32(0), key)
            keys_v[pl.ds(base + u * L, L)] = key
        return carry
    lax.fori_loop(0, S // (L * _UNROLL), kbody, jnp.int32(0))

    def count_ge(scand):
        def cb(i, cv):
            base = i * (L * _UNROLL)
            for u in range(_UNROLL):
                kv = keys_v[pl.ds(base + u * L, L)]
                cv = cv + jnp.where(kv >= scand, 1, 0).astype(jnp.int32)
            return cv
        return jnp.sum(lax.fori_loop(0, S // (L * _UNROLL), cb,
                                     jnp.zeros((L,), jnp.int32)))

    def sbody(j, t):
        cand = t | lax.shift_left(jnp.int32(1), jnp.int32(31) - j)
        cnt = count_ge(cand ^ _MIN32)
        return jnp.where(cnt >= K, cand, t)
    tbits = lax.fori_loop(0, 32, sbody, jnp.int32(0))
    thr = tbits ^ _MIN32

    def gtcb(i, cv):
        base = i * (L * _UNROLL)
        for u in range(_UNROLL):
            kv = keys_v[pl.ds(base + u * L, L)]
            cv = cv + jnp.where(kv > thr, 1, 0).astype(jnp.int32)
        return cv
    cgt = jnp.sum(lax.fori_loop(0, S // (L * _UNROLL), gtcb,
                                jnp.zeros((L,), jnp.int32)))
    need = K - cgt

    lanes = lax.iota(jnp.int32, L)

    def cbody(i, carry):
        off, eqs = carry
        base = i * (L * 4)
        for u in range(4):
            kv = keys_v[pl.ds(base + u * L, L)]
            gt = kv > thr
            eq = kv == thr
            eqc = plsc.cumsum(jnp.where(eq, 1, 0).astype(jnp.int32))
            keep = gt | (eq & ((eqs + eqc - 1) < need))
            kc = plsc.cumsum(jnp.where(keep, 1, 0).astype(jnp.int32))
            pos = off + kc - 1
            plsc.store_scatter(idx_v, [pos], lanes + (base + u * L), mask=keep)
            off = off + jnp.sum(jnp.where(keep, 1, 0).astype(jnp.int32))
            eqs = eqs + jnp.sum(jnp.where(eq, 1, 0).astype(jnp.int32))
        return (off, eqs)
    lax.fori_loop(0, S // (L * 4), cbody, (jnp.int32(0), jnp.int32(0)))

    @pl.when(slot == 0)
    def _():
        def sgb(i, carry):
            base = i * (L * 8)
            for u in range(8):
                iv = idx_v[pl.ds(base + u * L, L)]
                outs_v[pl.ds(base + u * L, L)] = plsc.load_gather(
                    scores_v, [iv])
            return carry
        lax.fori_loop(0, K // (L * 8), sgb, jnp.int32(0))
        pltpu.sync_copy(outs_v, outs_hbm.at[b])

    base = slot * RPT
    out0 = b * K + base
    row_off = b * S
    gbufs = (gbuf0, gbuf1, gbuf2)
    gsems = (gsem0, gsem1, gsem2)
    osems = (osem0, osem1, osem2)
    nb = 3
    pend_g = [None] * nb
    pend_o = [None] * nb
    for t in range(NCHUNK):
        bi = t % nb
        if pend_o[bi] is not None:
            pend_o[bi].wait()
        giv = idx_v[pl.ds(base + t * CH, CH)] + row_off
        pend_g[bi] = pltpu.async_copy(x_hbm.at[giv], gbufs[bi], gsems[bi])
        if t >= 1:
            pj = (t - 1) % nb
            pend_g[pj].wait()
            pend_o[pj] = pltpu.async_copy(
                gbufs[pj], outx_hbm.at[pl.ds(out0 + (t - 1) * CH, CH)],
                osems[pj])
    lb = (NCHUNK - 1) % nb
    pend_g[lb].wait()
    pend_o[lb] = pltpu.async_copy(
        gbufs[lb], outx_hbm.at[pl.ds(out0 + (NCHUNK - 1) * CH, CH)], osems[lb])
    for o in pend_o:
        if o is not None:
            o.wait()


@functools.cache
def _sc_topk_gather():
    return pl.kernel(
        _sc_body,
        out_type=(jax.ShapeDtypeStruct((B * K, D), jnp.float32),
                  jax.ShapeDtypeStruct((B, K), jnp.float32)),
        mesh=plsc.VectorSubcoreMesh(core_axis_name="c", subcore_axis_name="s"),
        compiler_params=pltpu.CompilerParams(needs_layout_passes=False),
        scratch_types=[
            pltpu.VMEM((S,), jnp.float32),
            pltpu.VMEM((S,), jnp.int32),
            pltpu.VMEM((K,), jnp.int32),
            pltpu.VMEM((K,), jnp.float32),
            pltpu.VMEM((CH, D), jnp.float32),
            pltpu.VMEM((CH, D), jnp.float32),
            pltpu.VMEM((CH, D), jnp.float32),
            pltpu.SemaphoreType.DMA,
            pltpu.SemaphoreType.DMA,
            pltpu.SemaphoreType.DMA,
            pltpu.SemaphoreType.DMA,
            pltpu.SemaphoreType.DMA,
            pltpu.SemaphoreType.DMA,
        ],
    )


def kernel(x, W1, b1, W2, b2):
    x2d = x.reshape(B * S, D)
    W1b = W1.astype(jnp.bfloat16)
    W2pb = jnp.pad(W2, ((0, 127), (0, 0))).astype(jnp.bfloat16)
    scores = _scores_tc(x2d, W1b, b1.reshape(1, H), W2pb, b2.reshape(1, 1))
    sliced_x, sliced_scores = _sc_topk_gather()(scores, x2d)
    return sliced_x.reshape(B, K, D), sliced_scores
